# baseline (speedup 1.0000x reference)
import functools

import jax
import jax.numpy as jnp
from jax import lax
from jax.experimental import pallas as pl
from jax.experimental.pallas import tpu as pltpu
from jax.experimental.pallas import tpu_sc as plsc

TOKENS = 16384
D_MODEL = 4096
N_NEURONS = 64
TOP_K = 8
BLK = 1024

_NC = 2
_NS = 16
_NW = _NC * _NS
T_TILE = TOKENS // _NW


def _logits_body(x_ref, w_ref, b_ref, out_ref):
    out_ref[...] = (
        jax.lax.dot_general(
            x_ref[...], w_ref[...], (((1,), (1,)), ((), ())),
            preferred_element_type=jnp.float32,
        )
        + b_ref[...]
    )


def _tc_logits(x, W, b):
    return pl.pallas_call(
        _logits_body,
        grid=(TOKENS // BLK,),
        in_specs=[
            pl.BlockSpec((BLK, D_MODEL), lambda i: (i, 0)),
            pl.BlockSpec((N_NEURONS, D_MODEL), lambda i: (0, 0)),
            pl.BlockSpec((1, N_NEURONS), lambda i: (0, 0)),
        ],
        out_specs=pl.BlockSpec((BLK, N_NEURONS), lambda i: (i, 0)),
        out_shape=jax.ShapeDtypeStruct((TOKENS, N_NEURONS), jnp.float32),
    )(x, W, b.reshape(1, N_NEURONS))


_GDN = lax.GatherDimensionNumbers(
    offset_dims=(), collapsed_slice_dims=(0,), start_index_map=(0,)
)


def _gat(x, idx):
    return lax.gather(
        x, idx[:, None], _GDN, (1,),
        mode=lax.GatherScatterMode.PROMISE_IN_BOUNDS,
    )


def _sc_topk_body(lg_hbm, gates_hbm, idx_hbm, lg_v, g_v, i_v):
    c = lax.axis_index("c")
    s = lax.axis_index("s")
    wid = s * _NC + c
    base = wid * (T_TILE * N_NEURONS)

    pltpu.sync_copy(lg_hbm.at[pl.ds(base, T_TILE * N_NEURONS)], lg_v)

    lane = lax.iota(jnp.int32, 16)
    lt8 = lane < 8
    gidx = jnp.maximum(lane - 8, 0)
    zeros = jnp.zeros((16,), jnp.int32)

    def merge8(ka, va, kb, vb):
        kc = jnp.where(lt8, ka, _gat(kb, gidx))
        vc = jnp.where(lt8, va, _gat(vb, gidx))
        return plsc.sort_key_val(kc, vc, descending=True)

    def token_top8(off):
        ks, vs = [], []
        for k in range(4):
            key = lg_v[pl.ds(off + 16 * k, 16)]
            kk, vv = plsc.sort_key_val(key, lane + 16 * k, descending=True)
            ks.append(kk)
            vs.append(vv)
        k01, v01 = merge8(ks[0], vs[0], ks[1], vs[1])
        k23, v23 = merge8(ks[2], vs[2], ks[3], vs[3])
        kf, vf = merge8(k01, v01, k23, v23)
        m = _gat(kf, zeros)
        e = jnp.where(lt8, jnp.exp(kf - m), 0.0)
        denom = jnp.sum(e, axis=0)
        return e / denom, vf

    def pair_body(p, carry):
        off = p * (2 * N_NEURONS)
        g0, i0 = token_top8(off)
        g1, i1 = token_top8(off + N_NEURONS)
        gm = jnp.where(lt8, g0, _gat(g1, gidx))
        im = jnp.where(lt8, i0, _gat(i1, gidx))
        g_v[pl.ds(p * 16, 16)] = gm
        i_v[pl.ds(p * 16, 16)] = im
        return carry

    lax.fori_loop(0, T_TILE // 2, pair_body, 0)

    obase = wid * (T_TILE * TOP_K)
    pltpu.sync_copy(g_v, gates_hbm.at[pl.ds(obase, T_TILE * TOP_K)])
    pltpu.sync_copy(i_v, idx_hbm.at[pl.ds(obase, T_TILE * TOP_K)])


@functools.partial(jax.jit, static_argnames=())
def kernel(x, W, b):
    logits = _tc_logits(x, W, b)

    sc = functools.partial(
        pl.kernel,
        mesh=plsc.VectorSubcoreMesh(core_axis_name="c", subcore_axis_name="s"),
        out_type=[
            jax.ShapeDtypeStruct((TOKENS * TOP_K,), jnp.float32),
            jax.ShapeDtypeStruct((TOKENS * TOP_K,), jnp.int32),
        ],
        scratch_types=[
            pltpu.VMEM((T_TILE * N_NEURONS,), jnp.float32),
            pltpu.VMEM((T_TILE * TOP_K,), jnp.float32),
            pltpu.VMEM((T_TILE * TOP_K,), jnp.int32),
        ],
        compiler_params=pltpu.CompilerParams(needs_layout_passes=False),
    )(_sc_topk_body)

    gates_f, idx_f = sc(logits.reshape(-1))
    return gates_f.reshape(TOKENS, TOP_K), idx_f.reshape(TOKENS, TOP_K)


# speedup vs baseline: 1.0409x; 1.0409x over previous
import functools

import jax
import jax.numpy as jnp
from jax import lax
from jax.experimental import pallas as pl
from jax.experimental.pallas import tpu as pltpu
from jax.experimental.pallas import tpu_sc as plsc

TOKENS = 16384
D_MODEL = 4096
N_NEURONS = 64
TOP_K = 8
BLK = 1024

_NC = 2
_NS = 16
_NW = _NC * _NS
T_TILE = TOKENS // _NW


def _logits_body(x_ref, w_ref, b_ref, out_ref):
    out_ref[...] = (
        jax.lax.dot_general(
            x_ref[...], w_ref[...], (((1,), (1,)), ((), ())),
            preferred_element_type=jnp.float32,
        )
        + b_ref[...]
    )


def _tc_logits(x, W, b):
    return pl.pallas_call(
        _logits_body,
        grid=(TOKENS // BLK,),
        in_specs=[
            pl.BlockSpec((BLK, D_MODEL), lambda i: (i, 0)),
            pl.BlockSpec((N_NEURONS, D_MODEL), lambda i: (0, 0)),
            pl.BlockSpec((1, N_NEURONS), lambda i: (0, 0)),
        ],
        out_specs=pl.BlockSpec((BLK, N_NEURONS), lambda i: (i, 0)),
        out_shape=jax.ShapeDtypeStruct((TOKENS, N_NEURONS), jnp.float32),
    )(x, W, b.reshape(1, N_NEURONS))


_GDN = lax.GatherDimensionNumbers(
    offset_dims=(), collapsed_slice_dims=(0,), start_index_map=(0,)
)


def _gat(x, idx):
    return lax.gather(
        x, idx[:, None], _GDN, (1,),
        mode=lax.GatherScatterMode.PROMISE_IN_BOUNDS,
    )


def _sc_topk_body(lg_hbm, gates_hbm, idx_hbm, lg_v, g_v, i_v, *, t_tile):
    c = lax.axis_index("c")
    s = lax.axis_index("s")
    wid = s * _NC + c
    base = wid * (t_tile * N_NEURONS)

    pltpu.sync_copy(lg_hbm.at[pl.ds(base, t_tile * N_NEURONS)], lg_v)

    lane = lax.iota(jnp.int32, 16)
    lt8 = lane < 8
    gidx = jnp.maximum(lane - 8, 0)
    zeros = jnp.zeros((16,), jnp.int32)

    def merge8(ka, va, kb, vb):
        kc = jnp.where(lt8, ka, _gat(kb, gidx))
        vc = jnp.where(lt8, va, _gat(vb, gidx))
        return plsc.sort_key_val(kc, vc, descending=True)

    def token_top8(off):
        ks, vs = [], []
        for k in range(4):
            key = lg_v[pl.ds(off + 16 * k, 16)]
            kk, vv = plsc.sort_key_val(key, lane + 16 * k, descending=True)
            ks.append(kk)
            vs.append(vv)
        k01, v01 = merge8(ks[0], vs[0], ks[1], vs[1])
        k23, v23 = merge8(ks[2], vs[2], ks[3], vs[3])
        kf, vf = merge8(k01, v01, k23, v23)
        m = _gat(kf, zeros)
        e = jnp.where(lt8, jnp.exp(kf - m), 0.0)
        denom = jnp.sum(e, axis=0)
        return e / denom, vf

    def pair_body(p, carry):
        off = p * (2 * N_NEURONS)
        g0, i0 = token_top8(off)
        g1, i1 = token_top8(off + N_NEURONS)
        gm = jnp.where(lt8, g0, _gat(g1, gidx))
        im = jnp.where(lt8, i0, _gat(i1, gidx))
        g_v[pl.ds(p * 16, 16)] = gm
        i_v[pl.ds(p * 16, 16)] = im
        return carry

    lax.fori_loop(0, t_tile // 2, pair_body, 0)

    obase = wid * (t_tile * TOP_K)
    pltpu.sync_copy(g_v, gates_hbm.at[pl.ds(obase, t_tile * TOP_K)])
    pltpu.sync_copy(i_v, idx_hbm.at[pl.ds(obase, t_tile * TOP_K)])


N_CHUNKS = 4
T_CHUNK = TOKENS // N_CHUNKS


def _tc_logits_chunk(x, W, b, ci):
    off = ci * (T_CHUNK // BLK)
    return pl.pallas_call(
        _logits_body,
        grid=(T_CHUNK // BLK,),
        in_specs=[
            pl.BlockSpec((BLK, D_MODEL), lambda i: (i + off, 0)),
            pl.BlockSpec((N_NEURONS, D_MODEL), lambda i: (0, 0)),
            pl.BlockSpec((1, N_NEURONS), lambda i: (0, 0)),
        ],
        out_specs=pl.BlockSpec((BLK, N_NEURONS), lambda i: (i, 0)),
        out_shape=jax.ShapeDtypeStruct((T_CHUNK, N_NEURONS), jnp.float32),
    )(x, W, b.reshape(1, N_NEURONS))


@functools.partial(jax.jit, static_argnames=())
def kernel(x, W, b):
    tct = T_CHUNK // _NW

    sc = functools.partial(
        pl.kernel,
        mesh=plsc.VectorSubcoreMesh(core_axis_name="c", subcore_axis_name="s"),
        out_type=[
            jax.ShapeDtypeStruct((T_CHUNK * TOP_K,), jnp.float32),
            jax.ShapeDtypeStruct((T_CHUNK * TOP_K,), jnp.int32),
        ],
        scratch_types=[
            pltpu.VMEM((tct * N_NEURONS,), jnp.float32),
            pltpu.VMEM((tct * TOP_K,), jnp.float32),
            pltpu.VMEM((tct * TOP_K,), jnp.int32),
        ],
        compiler_params=pltpu.CompilerParams(needs_layout_passes=False),
    )(functools.partial(_sc_topk_body, t_tile=tct))

    gs, is_ = [], []
    for ci in range(N_CHUNKS):
        logits = _tc_logits_chunk(x, W, b, ci)
        g, i = sc(logits.reshape(-1))
        gs.append(g.reshape(T_CHUNK, TOP_K))
        is_.append(i.reshape(T_CHUNK, TOP_K))
    return jnp.concatenate(gs, axis=0), jnp.concatenate(is_, axis=0)


# speedup vs baseline: 1.0449x; 1.0038x over previous
import functools

import jax
import jax.numpy as jnp
from jax import lax
from jax.experimental import pallas as pl
from jax.experimental.pallas import tpu as pltpu
from jax.experimental.pallas import tpu_sc as plsc

TOKENS = 16384
D_MODEL = 4096
N_NEURONS = 64
TOP_K = 8
BLK = 1024

_NC = 2
_NS = 16
_NW = _NC * _NS
T_TILE = TOKENS // _NW


def _logits_body(x_ref, w_ref, b_ref, out_ref):
    out_ref[...] = (
        jax.lax.dot_general(
            x_ref[...], w_ref[...], (((1,), (1,)), ((), ())),
            preferred_element_type=jnp.float32,
        )
        + b_ref[...]
    )


def _tc_logits(x, W, b):
    return pl.pallas_call(
        _logits_body,
        grid=(TOKENS // BLK,),
        in_specs=[
            pl.BlockSpec((BLK, D_MODEL), lambda i: (i, 0)),
            pl.BlockSpec((N_NEURONS, D_MODEL), lambda i: (0, 0)),
            pl.BlockSpec((1, N_NEURONS), lambda i: (0, 0)),
        ],
        out_specs=pl.BlockSpec((BLK, N_NEURONS), lambda i: (i, 0)),
        out_shape=jax.ShapeDtypeStruct((TOKENS, N_NEURONS), jnp.float32),
    )(x, W, b.reshape(1, N_NEURONS))


_GDN = lax.GatherDimensionNumbers(
    offset_dims=(), collapsed_slice_dims=(0,), start_index_map=(0,)
)


def _gat(x, idx):
    return lax.gather(
        x, idx[:, None], _GDN, (1,),
        mode=lax.GatherScatterMode.PROMISE_IN_BOUNDS,
    )


def _sc_topk_body(lg_hbm, gates_hbm, idx_hbm, lg_v, g_v, i_v, *, t_tile):
    c = lax.axis_index("c")
    s = lax.axis_index("s")
    wid = s * _NC + c
    base = wid * (t_tile * N_NEURONS)

    pltpu.sync_copy(lg_hbm.at[pl.ds(base, t_tile * N_NEURONS)], lg_v)

    lane = lax.iota(jnp.int32, 16)
    lt8 = lane < 8
    gidx = jnp.maximum(lane - 8, 0)
    zeros = jnp.zeros((16,), jnp.int32)

    def merge8(ka, va, kb, vb):
        kc = jnp.where(lt8, ka, _gat(kb, gidx))
        vc = jnp.where(lt8, va, _gat(vb, gidx))
        return plsc.sort_key_val(kc, vc, descending=True)

    def token_top8(off):
        ks, vs = [], []
        for k in range(4):
            key = lg_v[pl.ds(off + 16 * k, 16)]
            kk, vv = plsc.sort_key_val(key, lane + 16 * k, descending=True)
            ks.append(kk)
            vs.append(vv)
        k01, v01 = merge8(ks[0], vs[0], ks[1], vs[1])
        k23, v23 = merge8(ks[2], vs[2], ks[3], vs[3])
        kf, vf = merge8(k01, v01, k23, v23)
        m = _gat(kf, zeros)
        e = jnp.where(lt8, jnp.exp(kf - m), 0.0)
        denom = jnp.sum(e, axis=0)
        return e / denom, vf

    def pair_body(p, carry):
        off = p * (2 * N_NEURONS)
        g0, i0 = token_top8(off)
        g1, i1 = token_top8(off + N_NEURONS)
        gm = jnp.where(lt8, g0, _gat(g1, gidx))
        im = jnp.where(lt8, i0, _gat(i1, gidx))
        g_v[pl.ds(p * 16, 16)] = gm
        i_v[pl.ds(p * 16, 16)] = im
        return carry

    lax.fori_loop(0, t_tile // 2, pair_body, 0)

    obase = wid * (t_tile * TOP_K)
    pltpu.sync_copy(g_v, gates_hbm.at[pl.ds(obase, t_tile * TOP_K)])
    pltpu.sync_copy(i_v, idx_hbm.at[pl.ds(obase, t_tile * TOP_K)])


N_CHUNKS = 4
T_CHUNK = TOKENS // N_CHUNKS


def _tc_logits_chunk(x, W, b, ci):
    off = ci * (T_CHUNK // BLK)
    return pl.pallas_call(
        _logits_body,
        grid=(T_CHUNK // BLK,),
        in_specs=[
            pl.BlockSpec((BLK, D_MODEL), lambda i: (i + off, 0)),
            pl.BlockSpec((N_NEURONS, D_MODEL), lambda i: (0, 0)),
            pl.BlockSpec((1, N_NEURONS), lambda i: (0, 0)),
        ],
        out_specs=pl.BlockSpec((BLK, N_NEURONS), lambda i: (i, 0)),
        out_shape=jax.ShapeDtypeStruct((T_CHUNK, N_NEURONS), jnp.float32),
    )(x, W, b.reshape(1, N_NEURONS))


@functools.partial(jax.jit, static_argnames=())
def kernel(x, W, b):
    tct = T_CHUNK // _NW

    sc = functools.partial(
        pl.kernel,
        mesh=plsc.VectorSubcoreMesh(core_axis_name="c", subcore_axis_name="s"),
        out_type=[
            jax.ShapeDtypeStruct((T_CHUNK * TOP_K,), jnp.float32),
            jax.ShapeDtypeStruct((T_CHUNK * TOP_K,), jnp.int32),
        ],
        scratch_types=[
            pltpu.VMEM((tct * N_NEURONS,), jnp.float32),
            pltpu.VMEM((tct * TOP_K,), jnp.float32),
            pltpu.VMEM((tct * TOP_K,), jnp.int32),
        ],
        compiler_params=pltpu.CompilerParams(needs_layout_passes=False),
    )(functools.partial(_sc_topk_body, t_tile=tct))

    gs, is_ = [], []
    prev = _tc_logits_chunk(x, W, b, 0)
    for ci in range(1, N_CHUNKS):
        nxt = _tc_logits_chunk(x, W, b, ci)
        g, i = sc(prev.reshape(-1))
        gs.append(g.reshape(T_CHUNK, TOP_K))
        is_.append(i.reshape(T_CHUNK, TOP_K))
        prev = nxt
    g, i = sc(prev.reshape(-1))
    gs.append(g.reshape(T_CHUNK, TOP_K))
    is_.append(i.reshape(T_CHUNK, TOP_K))
    return jnp.concatenate(gs, axis=0), jnp.concatenate(is_, axis=0)


# speedup vs baseline: 1.5225x; 1.4572x over previous
import functools

import jax
import jax.numpy as jnp
from jax.experimental import pallas as pl
from jax.experimental.pallas import tpu as pltpu

TOKENS = 16384
D_MODEL = 4096
N_NEURONS = 64
TOP_K = 8
BLK = 1024
D_HALF = D_MODEL // 2


def _router_body(x0_ref, x1_ref, w_ref, b_ref, gates_ref, idx_ref):
    w = w_ref[...]
    dn = (((1,), (1,)), ((), ()))
    logits = jax.lax.dot_general(
        w[:, :D_HALF], x0_ref[...], dn, preferred_element_type=jnp.float32
    ) + jax.lax.dot_general(
        w[:, D_HALF:], x1_ref[...], dn, preferred_element_type=jnp.float32
    )
    logits = logits + b_ref[...]

    iota_f = jax.lax.broadcasted_iota(jnp.int32, logits.shape, 0).astype(jnp.float32)
    neg_inf = jnp.float32(-jnp.inf)
    sentinel = jnp.float32(N_NEURONS)

    vals = []
    idxs = []
    l = logits
    for _ in range(TOP_K):
        m = jnp.max(l, axis=0, keepdims=True)
        cand = jnp.where(l == m, iota_f, sentinel)
        am = jnp.min(cand, axis=0, keepdims=True)
        vals.append(m)
        idxs.append(am)
        l = jnp.where(cand == am, neg_inf, l)

    v = jnp.concatenate(vals, axis=0)
    e = jnp.exp(v - v[0:1])
    g = e / jnp.sum(e, axis=0, keepdims=True)
    idx_f = jnp.concatenate(idxs, axis=0)

    gates_ref[...] = g.T
    idx_ref[...] = idx_f.T.astype(jnp.int32)


@functools.partial(jax.jit, static_argnames=())
def kernel(x, W, b):
    grid = (TOKENS // BLK,)
    gates, idx = pl.pallas_call(
        _router_body,
        grid=grid,
        in_specs=[
            pl.BlockSpec((BLK, D_HALF), lambda i: (i, 0)),
            pl.BlockSpec((BLK, D_HALF), lambda i: (i, 1)),
            pl.BlockSpec((N_NEURONS, D_MODEL), lambda i: (0, 0)),
            pl.BlockSpec((N_NEURONS, 1), lambda i: (0, 0)),
        ],
        out_specs=[
            pl.BlockSpec((BLK, TOP_K), lambda i: (i, 0)),
            pl.BlockSpec((BLK, TOP_K), lambda i: (i, 0)),
        ],
        out_shape=[
            jax.ShapeDtypeStruct((TOKENS, TOP_K), jnp.float32),
            jax.ShapeDtypeStruct((TOKENS, TOP_K), jnp.int32),
        ],
    )(x, x, W, b.reshape(N_NEURONS, 1))
    return gates, idx
